# baseline (speedup 1.0000x reference)
import jax
import jax.numpy as jnp
from jax.experimental import pallas as pl
from jax.experimental.pallas import tpu as pltpu

E = 8
F_IN = 128
H1 = 64
H2 = 64
F_OUT = 16
B = 2048


def _celu(x):
    return jnp.where(x > 0, x, jnp.exp(jnp.minimum(x, 0.0)) - 1.0)


def _mlp_block_kernel(el_ref, x_ref, w1_ref, b1_ref, w2_ref, b2_ref, w3_ref, b3_ref, o_ref):
    el = el_ref[0]
    xT = jnp.transpose(x_ref[...], (1, 0)).astype(jnp.bfloat16)
    masks = [el == e for e in range(E)]

    def layer(hT, w_ref, b_ref, width):
        zall = jax.lax.dot_general(w_ref[...].astype(jnp.bfloat16), hT,
                                   (((1,), (0,)), ((), ())),
                                   preferred_element_type=jnp.float32)
        zall = zall + b_ref[...]
        z = jnp.where(masks[0], zall[:width], jnp.zeros((width, hT.shape[1]), jnp.float32))
        for e in range(1, E):
            z = jnp.where(masks[e], zall[e * width:(e + 1) * width], z)
        return z

    h = _celu(layer(xT, w1_ref, b1_ref, H1)).astype(jnp.bfloat16)
    h = _celu(layer(h, w2_ref, b2_ref, H2)).astype(jnp.bfloat16)
    z3 = layer(h, w3_ref, b3_ref, F_OUT)
    z3p = jnp.concatenate(
        [z3, jnp.zeros((F_IN - F_OUT, z3.shape[1]), jnp.float32)], axis=0)
    o_ref[...] = jnp.transpose(z3p, (1, 0))[:, :F_OUT]


def kernel(elements, features, W1, b1, W2, b2, W3, b3):
    n, M, f = features.shape
    N = n * M
    nblk = N // B
    x = features.reshape(N, f)
    el3 = elements.reshape(nblk, 1, B)

    grid_spec = pl.GridSpec(
        grid=(nblk,),
        in_specs=[
            pl.BlockSpec((1, 1, B), lambda i: (i, 0, 0)),
            pl.BlockSpec((B, F_IN), lambda i: (i, 0)),
            pl.BlockSpec((E * H1, F_IN), lambda i: (0, 0)),
            pl.BlockSpec((E * H1, 1), lambda i: (0, 0)),
            pl.BlockSpec((E * H2, H1), lambda i: (0, 0)),
            pl.BlockSpec((E * H2, 1), lambda i: (0, 0)),
            pl.BlockSpec((E * F_OUT, H2), lambda i: (0, 0)),
            pl.BlockSpec((E * F_OUT, 1), lambda i: (0, 0)),
        ],
        out_specs=pl.BlockSpec((B, F_OUT), lambda i: (i, 0)),
    )
    y = pl.pallas_call(
        _mlp_block_kernel,
        grid_spec=grid_spec,
        out_shape=jax.ShapeDtypeStruct((N, F_OUT), jnp.float32),
        compiler_params=pltpu.CompilerParams(
            dimension_semantics=("parallel",)),
    )(el3, x, W1.reshape(E * H1, F_IN), b1.reshape(E * H1, 1),
      W2.reshape(E * H2, H1), b2.reshape(E * H2, 1),
      W3.reshape(E * F_OUT, H2), b3.reshape(E * F_OUT, 1))
    return (elements, y.reshape(n, M, F_OUT))


# speedup vs baseline: 1.0478x; 1.0478x over previous
import jax
import jax.numpy as jnp
from jax.experimental import pallas as pl
from jax.experimental.pallas import tpu as pltpu

E = 8
F_IN = 128
H1 = 64
H2 = 64
F_OUT = 16
B = 4096


def _celu(x):
    return jnp.where(x > 0, x, jnp.exp(jnp.minimum(x, 0.0)) - 1.0)


def _mlp_block_kernel(el_ref, x_ref, w1_ref, b1_ref, w2_ref, b2_ref, w3_ref, b3_ref, o_ref):
    el = el_ref[0]
    xT = jnp.transpose(x_ref[...], (1, 0)).astype(jnp.bfloat16)
    masks = [el == e for e in range(E)]

    def layer(hT, w_ref, b_ref, width):
        zall = jax.lax.dot_general(w_ref[...].astype(jnp.bfloat16), hT,
                                   (((1,), (0,)), ((), ())),
                                   preferred_element_type=jnp.float32)
        zall = zall + b_ref[...]
        z = jnp.where(masks[0], zall[:width], jnp.zeros((width, hT.shape[1]), jnp.float32))
        for e in range(1, E):
            z = jnp.where(masks[e], zall[e * width:(e + 1) * width], z)
        return z

    h = _celu(layer(xT, w1_ref, b1_ref, H1)).astype(jnp.bfloat16)
    h = _celu(layer(h, w2_ref, b2_ref, H2)).astype(jnp.bfloat16)
    z3 = layer(h, w3_ref, b3_ref, F_OUT)
    z3p = jnp.concatenate(
        [z3, jnp.zeros((F_IN - F_OUT, z3.shape[1]), jnp.float32)], axis=0)
    o_ref[...] = jnp.transpose(z3p, (1, 0))[:, :F_OUT]


def kernel(elements, features, W1, b1, W2, b2, W3, b3):
    n, M, f = features.shape
    N = n * M
    nblk = N // B
    x = features.reshape(N, f)
    el3 = elements.reshape(nblk, 1, B)

    grid_spec = pl.GridSpec(
        grid=(nblk,),
        in_specs=[
            pl.BlockSpec((1, 1, B), lambda i: (i, 0, 0)),
            pl.BlockSpec((B, F_IN), lambda i: (i, 0)),
            pl.BlockSpec((E * H1, F_IN), lambda i: (0, 0)),
            pl.BlockSpec((E * H1, 1), lambda i: (0, 0)),
            pl.BlockSpec((E * H2, H1), lambda i: (0, 0)),
            pl.BlockSpec((E * H2, 1), lambda i: (0, 0)),
            pl.BlockSpec((E * F_OUT, H2), lambda i: (0, 0)),
            pl.BlockSpec((E * F_OUT, 1), lambda i: (0, 0)),
        ],
        out_specs=pl.BlockSpec((B, F_OUT), lambda i: (i, 0)),
    )
    y = pl.pallas_call(
        _mlp_block_kernel,
        grid_spec=grid_spec,
        out_shape=jax.ShapeDtypeStruct((N, F_OUT), jnp.float32),
        compiler_params=pltpu.CompilerParams(
            dimension_semantics=("parallel",)),
    )(el3, x, W1.reshape(E * H1, F_IN), b1.reshape(E * H1, 1),
      W2.reshape(E * H2, H1), b2.reshape(E * H2, 1),
      W3.reshape(E * F_OUT, H2), b3.reshape(E * F_OUT, 1))
    return (elements, y.reshape(n, M, F_OUT))
